# Initial kernel scaffold; baseline (speedup 1.0000x reference)
import jax
import jax.numpy as jnp
from jax import lax
from jax.experimental import pallas as pl
from jax.experimental.pallas import tpu as pltpu
from jax.experimental.pallas import tpu_sc as plsc

N = 10000
E = 320000
D = 128
GROUP = 128
G = E // GROUP
NC, NS, L = 2, 16, 16
NW = NC * NS
GPT = G // NW
REM = G - GPT * NW
GMAX = GPT + 1
RPS = N // NS


def _mesh():
    return plsc.VectorSubcoreMesh(
        core_axis_name="c", subcore_axis_name="s", num_cores=NC, num_subcores=NS
    )


def _worker(c, s):
    w = s * NC + c
    base = GPT * w + jnp.minimum(w, REM)
    cnt = GPT + (w < REM).astype(jnp.int32)
    return w, base, cnt


def _segsum_body(h_hbm, u_hbm, v_hbm, zero_hbm, out_hbm, u_v, v_v, rows, gsem, acc):
    c = lax.axis_index("c")
    s = lax.axis_index("s")
    _, base, cnt = _worker(c, s)
    pltpu.sync_copy(zero_hbm.at[pl.ds(s * RPS, RPS)], acc.at[pl.ds(s * RPS, RPS)])
    pltpu.sync_copy(u_hbm.at[pl.ds(base, GMAX)], u_v)
    pltpu.sync_copy(v_hbm.at[pl.ds(base, GMAX)], v_v)
    plsc.subcore_barrier()

    def grp(j, carry):
        pltpu.async_copy(h_hbm.at[u_v.at[j]], rows, gsem).wait()
        pltpu.sync_copy(rows, acc.at[v_v.at[j]], add=True)
        return carry

    lax.fori_loop(0, cnt, grp, 0)
    plsc.subcore_barrier()
    pltpu.sync_copy(acc.at[pl.ds(s * RPS, RPS)], out_hbm.at[c, pl.ds(s * RPS, RPS)])


def _segsum_ew_body(h_hbm, u_hbm, v_hbm, zero_hbm, a_hbm, b_hbm,
                    out_hbm, ew_hbm,
                    u_v, v_v, rows, ew_row, a_v, b_v, gsem, acc):
    c = lax.axis_index("c")
    s = lax.axis_index("s")
    _, base, cnt = _worker(c, s)
    pltpu.sync_copy(zero_hbm.at[pl.ds(s * RPS, RPS)], acc.at[pl.ds(s * RPS, RPS)])
    pltpu.sync_copy(u_hbm.at[pl.ds(base, GMAX)], u_v)
    pltpu.sync_copy(v_hbm.at[pl.ds(base, GMAX)], v_v)
    pltpu.sync_copy(a_hbm, a_v)
    pltpu.sync_copy(b_hbm, b_v)
    plsc.subcore_barrier()

    lanes = lax.iota(jnp.int32, L)

    def grp(j, carry):
        pltpu.async_copy(h_hbm.at[u_v.at[j]], rows, gsem).wait()
        pltpu.sync_copy(rows, acc.at[v_v.at[j]], add=True)
        jvec = jnp.full((L,), j, jnp.int32)
        for k in range(GROUP // L):
            lk = lanes + (k * L)
            u16 = plsc.load_gather(u_v, [jvec, lk])
            v16 = plsc.load_gather(v_v, [jvec, lk])
            bg = plsc.load_gather(b_v, [u16])
            ag = plsc.load_gather(a_v, [v16])
            ew_row[pl.ds(k * L, L)] = ag * bg
        pltpu.sync_copy(ew_row, ew_hbm.at[base + j])
        return carry

    lax.fori_loop(0, cnt, grp, 0)
    plsc.subcore_barrier()
    pltpu.sync_copy(acc.at[pl.ds(s * RPS, RPS)], out_hbm.at[c, pl.ds(s * RPS, RPS)])


def _sc_segsum(h, u2, v2, zeros):
    return pl.kernel(
        _segsum_body,
        out_type=jax.ShapeDtypeStruct((NC, N, D), jnp.float32),
        mesh=_mesh(),
        scratch_types=[
            pltpu.VMEM((GMAX, GROUP), jnp.int32),
            pltpu.VMEM((GMAX, GROUP), jnp.int32),
            pltpu.VMEM((GROUP, D), jnp.float32),
            pltpu.SemaphoreType.DMA,
            pltpu.VMEM_SHARED((N, D), jnp.float32),
        ],
    )(h, u2, v2, zeros)


def _sc_segsum_ew(h, u2, v2, zeros, a, b):
    return pl.kernel(
        _segsum_ew_body,
        out_type=(
            jax.ShapeDtypeStruct((NC, N, D), jnp.float32),
            jax.ShapeDtypeStruct((G, GROUP), jnp.float32),
        ),
        mesh=_mesh(),
        scratch_types=[
            pltpu.VMEM((GMAX, GROUP), jnp.int32),
            pltpu.VMEM((GMAX, GROUP), jnp.int32),
            pltpu.VMEM((GROUP, D), jnp.float32),
            pltpu.VMEM((GROUP,), jnp.float32),
            pltpu.VMEM((N,), jnp.float32),
            pltpu.VMEM((N,), jnp.float32),
            pltpu.SemaphoreType.DMA,
            pltpu.VMEM_SHARED((N, D), jnp.float32),
        ],
    )(h, u2, v2, zeros, a, b)


def _enc_body(x_ref, w_ref, b_ref, o_ref):
    o_ref[...] = (
        jnp.dot(x_ref[...], w_ref[...], preferred_element_type=jnp.float32)
        + b_ref[...]
    )


def _decide_body(h_ref, p_ref, iw_ref, ib_ref, ow_ref, ob_ref, g1_ref, g2_ref,
                 a_ref, b_ref, hb_ref):
    h = h_ref[...]
    agg = p_ref[0] + p_ref[1]
    cat = jnp.concatenate([h, agg], axis=-1)
    z1 = jnp.dot(cat, iw_ref[...], preferred_element_type=jnp.float32) + ib_ref[...] + g1_ref[...]
    z2 = jnp.dot(cat, ow_ref[...], preferred_element_type=jnp.float32) + ob_ref[...] + g2_ref[...]
    a = (z1[:, 0] >= z1[:, 1]).astype(jnp.float32)
    b = (z2[:, 0] >= z2[:, 1]).astype(jnp.float32)
    a_ref[...] = a
    b_ref[...] = b
    hb_ref[...] = h * b[:, None]


def _conv_body(h_ref, p_ref, a_ref, w_ref, b_ref, o_ref):
    agg = (p_ref[0] + p_ref[1]) * a_ref[...][:, None]
    cat = jnp.concatenate([h_ref[...], agg], axis=-1)
    o_ref[...] = (
        jnp.dot(cat, w_ref[...], preferred_element_type=jnp.float32) + b_ref[...]
    )


def _final_body(h_ref, p_ref, a_ref, w_ref, b_ref, fw_ref, fb_ref, o_ref):
    agg = (p_ref[0] + p_ref[1]) * a_ref[...][:, None]
    cat = jnp.concatenate([h_ref[...], agg], axis=-1)
    h2 = jnp.dot(cat, w_ref[...], preferred_element_type=jnp.float32) + b_ref[...]
    o_ref[...] = (
        jnp.dot(h2, fw_ref[...], preferred_element_type=jnp.float32) + fb_ref[...]
    )


def _tc_encode(x, w, b):
    return pl.pallas_call(
        _enc_body,
        out_shape=jax.ShapeDtypeStruct((N, D), jnp.float32),
    )(x, w, b)


def _tc_decide(h, parts, iw, ib, ow, ob, g1, g2):
    return pl.pallas_call(
        _decide_body,
        out_shape=(
            jax.ShapeDtypeStruct((N,), jnp.float32),
            jax.ShapeDtypeStruct((N,), jnp.float32),
            jax.ShapeDtypeStruct((N, D), jnp.float32),
        ),
    )(h, parts, iw, ib, ow, ob, g1, g2)


def _tc_conv(h, parts, a, w, b):
    return pl.pallas_call(
        _conv_body,
        out_shape=jax.ShapeDtypeStruct((N, D), jnp.float32),
    )(h, parts, a, w, b)


def _tc_final(h, parts, a, w, b, fw, fb):
    return pl.pallas_call(
        _final_body,
        out_shape=jax.ShapeDtypeStruct((N, D), jnp.float32),
    )(h, parts, a, w, b, fw, fb)


def kernel(x, edge_index, enc_W, enc_b, conv1_W, conv1_b, conv2_W, conv2_b,
           in_act_W, in_act_b, out_act_W, out_act_b, final_W, final_b):
    u2 = edge_index[0].astype(jnp.int32).reshape(G, GROUP)
    v2 = edge_index[1].astype(jnp.int32).reshape(G, GROUP)
    pad = ((0, NW), (0, 0))
    u2 = jnp.pad(u2, pad)
    v2 = jnp.pad(v2, pad)
    zeros = jnp.zeros((N, D), jnp.float32)

    base_key = jax.random.key(42)
    gumbels = []
    for i in range(2):
        k1, k2 = jax.random.split(jax.random.fold_in(base_key, i))
        gumbels.append((
            jax.random.gumbel(k1, (N, 2), jnp.float32),
            jax.random.gumbel(k2, (N, 2), jnp.float32),
        ))

    h = _tc_encode(x, enc_W, enc_b)

    parts = _sc_segsum(h, u2, v2, zeros)
    a0, b0, hb0 = _tc_decide(h, parts, in_act_W, in_act_b, out_act_W, out_act_b,
                             *gumbels[0])
    parts_w, ew0 = _sc_segsum_ew(hb0, u2, v2, zeros, a0, b0)
    h = _tc_conv(h, parts_w, a0, conv1_W, conv1_b)

    parts = _sc_segsum(h, u2, v2, zeros)
    a1, b1, hb1 = _tc_decide(h, parts, in_act_W, in_act_b, out_act_W, out_act_b,
                             *gumbels[1])
    parts_w, ew1 = _sc_segsum_ew(hb1, u2, v2, zeros, a1, b1)
    out = _tc_final(h, parts_w, a1, conv2_W, conv2_b, final_W, final_b)

    return (out, ew0.reshape(E), ew1.reshape(E))


# speedup vs baseline: 5.1318x; 5.1318x over previous
import jax
import jax.numpy as jnp
from jax import lax
from jax.experimental import pallas as pl
from jax.experimental.pallas import tpu as pltpu
from jax.experimental.pallas import tpu_sc as plsc

N = 10000
E = 320000
D = 128
GROUP = 128
G = E // GROUP
NC, NS, L = 1, 16, 16
NW = NC * NS
GMAX = 160
GPAD = GMAX * NW
NBLK = GMAX // 4
RPS = (N // NS) & ~7
TAIL = N - RPS * NS


def _mesh():
    return plsc.VectorSubcoreMesh(
        core_axis_name="c", subcore_axis_name="s", num_cores=NC, num_subcores=NS
    )


def _acc_zero(zero_hbm, acc, s):
    pltpu.sync_copy(zero_hbm.at[pl.ds(s * RPS, RPS)], acc.at[pl.ds(s * RPS, RPS)])

    @pl.when(s == NS - 1)
    def _():
        pltpu.sync_copy(zero_hbm.at[pl.ds(RPS * NS, TAIL)],
                        acc.at[pl.ds(RPS * NS, TAIL)])


def _acc_writeback(acc, out_hbm, c, s):
    pltpu.sync_copy(acc.at[pl.ds(s * RPS, RPS)],
                    out_hbm.at[c, pl.ds(s * RPS, RPS)])

    @pl.when(s == NS - 1)
    def _():
        pltpu.sync_copy(acc.at[pl.ds(RPS * NS, TAIL)],
                        out_hbm.at[c, pl.ds(RPS * NS, TAIL)])


def _make_segsum_body(with_ew):
    def body(*args):
        if with_ew:
            (h_hbm, u1_hbm, v1_hbm, zero_hbm, abp_hbm, out_hbm, ew_hbm,
             ur0, ur1, vr0, vr1, rows0, rows1, abp_v, ew0_v, ew1_v,
             usem0, usem1, vsem0, vsem1, gsem0, gsem1, ssem0, ssem1,
             esem0, esem1, acc) = args
            ews = (ew0_v, ew1_v)
            esems = (esem0, esem1)
        else:
            (h_hbm, u1_hbm, v1_hbm, zero_hbm, out_hbm,
             ur0, ur1, vr0, vr1, rows0, rows1,
             usem0, usem1, vsem0, vsem1, gsem0, gsem1, ssem0, ssem1,
             acc) = args
        urs = (ur0, ur1)
        vrs = (vr0, vr1)
        rows = (rows0, rows1)
        usems = (usem0, usem1)
        vsems = (vsem0, vsem1)
        gsems = (gsem0, gsem1)
        ssems = (ssem0, ssem1)

        c = lax.axis_index("c")
        s = lax.axis_index("s")
        w = s * NC + c
        base = w * GMAX

        _acc_zero(zero_hbm, acc, s)
        if with_ew:
            pltpu.sync_copy(abp_hbm, abp_v.at[pl.ds(0, N)])
        plsc.subcore_barrier()

        def idx_u(j, k):
            return pltpu.make_async_copy(
                u1_hbm.at[pl.ds((base + j) * GROUP, GROUP)], urs[k], usems[k])

        def idx_v(j, k):
            return pltpu.make_async_copy(
                v1_hbm.at[pl.ds((base + j) * GROUP, GROUP)], vrs[k], vsems[k])

        def gather(k, r):
            return pltpu.make_async_copy(h_hbm.at[urs[k]], rows[r], gsems[r])

        def scatter_start(k, r):
            return pltpu.async_copy(rows[r], acc.at[vrs[k]], ssems[r], add=True)

        def ew_store(j, k):
            return pltpu.make_async_copy(
                ews[k], ew_hbm.at[pl.ds((base + j) * GROUP, GROUP)], esems[k])

        def ew_compute(k):
            for t in range(GROUP // L):
                u16 = urs[k][pl.ds(t * L, L)]
                v16 = vrs[k][pl.ds(t * L, L)]
                gu = plsc.load_gather(abp_v, [u16])
                gv = plsc.load_gather(abp_v, [v16])
                b_u = (gu >= 2.0).astype(jnp.float32)
                a_v = gv - 2.0 * (gv >= 2.0).astype(jnp.float32)
                ews[k][pl.ds(t * L, L)] = a_v * b_u

        def slot(j, r, last):
            rn = 1 - r
            d_s = scatter_start(r, r)
            d_g = None
            d_e = None
            if not last:
                d_u = idx_u(j + 1, rn)
                d_v = idx_v(j + 1, rn)
                d_u.start()
                d_v.start()
            if with_ew:
                ew_compute(r)
                d_e = ew_store(j, r)
                d_e.start()
            if not last:
                d_u.wait()
                d_v.wait()
                d_g = gather(rn, rn)
                d_g.start()
            d_s.wait()
            if d_g is not None:
                d_g.wait()
            if d_e is not None:
                d_e.wait()

        d = idx_u(0, 0)
        d.start()
        d.wait()
        d = idx_v(0, 0)
        d.start()
        d.wait()
        d = gather(0, 0)
        d.start()
        d.wait()

        def two_slots(i, carry):
            j = 2 * i
            slot(j, 0, False)
            slot(j + 1, 1, False)
            return carry

        lax.fori_loop(0, GMAX // 2 - 1, two_slots, 0)

        slot(GMAX - 2, 0, False)
        slot(GMAX - 1, 1, True)

        plsc.subcore_barrier()
        _acc_writeback(acc, out_hbm, c, s)

    return body


_segsum_body = _make_segsum_body(False)
_segsum_ew_body = _make_segsum_body(True)


def _sc_segsum(h, u1, v1, zeros):
    return pl.kernel(
        _segsum_body,
        out_type=jax.ShapeDtypeStruct((NC, N, D), jnp.float32),
        mesh=_mesh(),
        compiler_params=pltpu.CompilerParams(needs_layout_passes=False),
        scratch_types=[
            pltpu.VMEM((GROUP,), jnp.int32),
            pltpu.VMEM((GROUP,), jnp.int32),
            pltpu.VMEM((GROUP,), jnp.int32),
            pltpu.VMEM((GROUP,), jnp.int32),
            pltpu.VMEM((GROUP, D), jnp.float32),
            pltpu.VMEM((GROUP, D), jnp.float32),
        ] + [pltpu.SemaphoreType.DMA] * 8 + [
            pltpu.VMEM_SHARED((N + 8, D), jnp.float32),
        ],
    )(h, u1, v1, zeros)


def _sc_segsum_ew(h, u1, v1, zeros, abp):
    return pl.kernel(
        _segsum_ew_body,
        out_type=(
            jax.ShapeDtypeStruct((NC, N, D), jnp.float32),
            jax.ShapeDtypeStruct((GPAD * GROUP,), jnp.float32),
        ),
        mesh=_mesh(),
        compiler_params=pltpu.CompilerParams(needs_layout_passes=False),
        scratch_types=[
            pltpu.VMEM((GROUP,), jnp.int32),
            pltpu.VMEM((GROUP,), jnp.int32),
            pltpu.VMEM((GROUP,), jnp.int32),
            pltpu.VMEM((GROUP,), jnp.int32),
            pltpu.VMEM((GROUP, D), jnp.float32),
            pltpu.VMEM((GROUP, D), jnp.float32),
            pltpu.VMEM((N + 16,), jnp.float32),
            pltpu.VMEM((GROUP,), jnp.float32),
            pltpu.VMEM((GROUP,), jnp.float32),
        ] + [pltpu.SemaphoreType.DMA] * 10 + [
            pltpu.VMEM_SHARED((N + 8, D), jnp.float32),
        ],
    )(h, u1, v1, zeros, abp)


def _parts_sum(p_ref):
    agg = p_ref[0]
    for i in range(1, NC):
        agg = agg + p_ref[i]
    return agg


def _enc_body(x_ref, w_ref, b_ref, o_ref):
    o_ref[...] = (
        jnp.dot(x_ref[...], w_ref[...], preferred_element_type=jnp.float32)
        + b_ref[...]
    )


def _decide_body(h_ref, p_ref, iw_ref, ib_ref, ow_ref, ob_ref,
                 g1a_ref, g1b_ref, g2a_ref, g2b_ref,
                 a_ref, abp_ref, hb_ref):
    h = h_ref[...]
    agg = _parts_sum(p_ref)
    cat = jnp.concatenate([h, agg], axis=-1)
    il = jnp.dot(cat, iw_ref[...], preferred_element_type=jnp.float32) + ib_ref[...]
    ol = jnp.dot(cat, ow_ref[...], preferred_element_type=jnp.float32) + ob_ref[...]
    a = (il[:, 0] + g1a_ref[...] >= il[:, 1] + g1b_ref[...]).astype(jnp.float32)
    b = (ol[:, 0] + g2a_ref[...] >= ol[:, 1] + g2b_ref[...]).astype(jnp.float32)
    a_ref[...] = a
    abp_ref[...] = a + 2.0 * b
    hb_ref[...] = h * b[:, None]


def _conv_body(h_ref, p_ref, a_ref, w_ref, b_ref, o_ref):
    agg = _parts_sum(p_ref) * a_ref[...][:, None]
    cat = jnp.concatenate([h_ref[...], agg], axis=-1)
    o_ref[...] = (
        jnp.dot(cat, w_ref[...], preferred_element_type=jnp.float32) + b_ref[...]
    )


def _final_body(h_ref, p_ref, a_ref, w_ref, b_ref, fw_ref, fb_ref, o_ref):
    agg = _parts_sum(p_ref) * a_ref[...][:, None]
    cat = jnp.concatenate([h_ref[...], agg], axis=-1)
    h2 = jnp.dot(cat, w_ref[...], preferred_element_type=jnp.float32) + b_ref[...]
    o_ref[...] = (
        jnp.dot(h2, fw_ref[...], preferred_element_type=jnp.float32) + fb_ref[...]
    )


def _tc_encode(x, w, b):
    return pl.pallas_call(
        _enc_body,
        out_shape=jax.ShapeDtypeStruct((N, D), jnp.float32),
    )(x, w, b)


def _tc_decide(h, parts, iw, ib, ow, ob, g1, g2):
    return pl.pallas_call(
        _decide_body,
        out_shape=(
            jax.ShapeDtypeStruct((N,), jnp.float32),
            jax.ShapeDtypeStruct((N,), jnp.float32),
            jax.ShapeDtypeStruct((N, D), jnp.float32),
        ),
    )(h, parts, iw, ib, ow, ob, g1[:, 0], g1[:, 1], g2[:, 0], g2[:, 1])


def _tc_conv(h, parts, a, w, b):
    return pl.pallas_call(
        _conv_body,
        out_shape=jax.ShapeDtypeStruct((N, D), jnp.float32),
    )(h, parts, a, w, b)


def _tc_final(h, parts, a, w, b, fw, fb):
    return pl.pallas_call(
        _final_body,
        out_shape=jax.ShapeDtypeStruct((N, D), jnp.float32),
    )(h, parts, a, w, b, fw, fb)


def kernel(x, edge_index, enc_W, enc_b, conv1_W, conv1_b, conv2_W, conv2_b,
           in_act_W, in_act_b, out_act_W, out_act_b, final_W, final_b):
    npad = (GPAD - G) * GROUP
    u1 = jnp.pad(edge_index[0].astype(jnp.int32), (0, npad))
    v1 = jnp.pad(edge_index[1].astype(jnp.int32), (0, npad), constant_values=N)
    zeros = jnp.zeros((N, D), jnp.float32)

    base_key = jax.random.key(42)
    gumbels = []
    for i in range(2):
        k1, k2 = jax.random.split(jax.random.fold_in(base_key, i))
        gumbels.append((
            jax.random.gumbel(k1, (N, 2), jnp.float32),
            jax.random.gumbel(k2, (N, 2), jnp.float32),
        ))

    h = _tc_encode(x, enc_W, enc_b)

    parts = _sc_segsum(h, u1, v1, zeros)
    a0, abp0, hb0 = _tc_decide(h, parts, in_act_W, in_act_b, out_act_W,
                               out_act_b, *gumbels[0])
    parts_w, ew0 = _sc_segsum_ew(hb0, u1, v1, zeros, abp0)
    h = _tc_conv(h, parts_w, a0, conv1_W, conv1_b)

    parts = _sc_segsum(h, u1, v1, zeros)
    a1, abp1, hb1 = _tc_decide(h, parts, in_act_W, in_act_b, out_act_W,
                               out_act_b, *gumbels[1])
    parts_w, ew1 = _sc_segsum_ew(hb1, u1, v1, zeros, abp1)
    out = _tc_final(h, parts_w, a1, conv2_W, conv2_b, final_W, final_b)

    return (out, ew0[:E], ew1[:E])
